# baseline (speedup 1.0000x reference)
import jax
import jax.numpy as jnp
from jax.experimental import pallas as pl
from jax.experimental.pallas import tpu as pltpu

_N = 5000
_D = 12544
_H = 1024
_O = 16
_TN = 512
_KT = 7
_DK = _D // _KT
_J = 4
_DJ = _DK // _J


def _boxhead_body(fv_ref, w1_ref, b1_ref, w2_ref, b2_ref, wh_ref, bh_ref,
                  out_ref, w1b_ref, w2b_ref, acc_ref):
    n = pl.program_id(0)
    k = pl.program_id(1)
    ksl = pl.ds(k * _DK, _DK)

    @pl.when(n == 0)
    def _cast_w1_block():
        w1b_ref[ksl, :] = w1_ref[...].astype(jnp.bfloat16)

    @pl.when(jnp.logical_and(n == 0, k == 0))
    def _cast_w2():
        w2b_ref[...] = w2_ref[...].astype(jnp.bfloat16)

    part = None
    for j in range(_J):
        js = pl.ds(j * _DJ, _DJ)
        a = fv_ref[:, js].astype(jnp.bfloat16)
        b = w1b_ref[pl.ds(k * _DK + j * _DJ, _DJ), :]
        p = jnp.dot(a, b, preferred_element_type=jnp.float32)
        part = p if part is None else part + p

    @pl.when(k == 0)
    def _init():
        acc_ref[...] = part

    @pl.when(jnp.logical_and(k > 0, k < _KT - 1))
    def _accum():
        acc_ref[...] = acc_ref[...] + part

    @pl.when(k == _KT - 1)
    def _finish():
        x = acc_ref[...] + part + b1_ref[...]
        x = jnp.maximum(x, 0.0).astype(jnp.bfloat16)
        x = jnp.dot(x, w2b_ref[...], preferred_element_type=jnp.float32)
        x = jnp.maximum(x + b2_ref[...], 0.0).astype(jnp.bfloat16)
        out_ref[...] = (
            jnp.dot(x, wh_ref[...].astype(jnp.bfloat16),
                    preferred_element_type=jnp.float32)
            + bh_ref[...]
        )


def kernel(feature_vectors, W1, b1, W2, b2, Wc, bc, Wr, br):
    Wh = jnp.concatenate([Wc, Wr], axis=1)
    bh = jnp.concatenate([bc, br])[None, :]
    out = pl.pallas_call(
        _boxhead_body,
        grid=(pl.cdiv(_N, _TN), _KT),
        in_specs=[
            pl.BlockSpec((_TN, _DK), lambda n, k: (n, k)),
            pl.BlockSpec((_DK, _H),
                         lambda n, k: (jnp.where(n == 0, k, 0), 0)),
            pl.BlockSpec((1, _H), lambda n, k: (0, 0)),
            pl.BlockSpec((_H, _H), lambda n, k: (0, 0)),
            pl.BlockSpec((1, _H), lambda n, k: (0, 0)),
            pl.BlockSpec((_H, _O), lambda n, k: (0, 0)),
            pl.BlockSpec((1, _O), lambda n, k: (0, 0)),
        ],
        out_specs=pl.BlockSpec((_TN, _O), lambda n, k: (n, 0)),
        out_shape=jax.ShapeDtypeStruct((_N, _O), jnp.float32),
        scratch_shapes=[
            pltpu.VMEM((_D, _H), jnp.bfloat16),
            pltpu.VMEM((_H, _H), jnp.bfloat16),
            pltpu.VMEM((_TN, _H), jnp.float32),
        ],
        compiler_params=pltpu.CompilerParams(
            vmem_limit_bytes=62 * 1024 * 1024),
    )(feature_vectors, W1, b1[None, :], W2, b2[None, :], Wh, bh)
    return out[:, :4], out[:, 4:]


# speedup vs baseline: 1.1089x; 1.1089x over previous
import jax
import jax.numpy as jnp
from jax.experimental import pallas as pl
from jax.experimental.pallas import tpu as pltpu

_N = 5000
_D = 12544
_H = 1024
_O = 16
_TN = 256


def _boxhead_body(fv_ref, w1_ref, b1_ref, w2_ref, b2_ref, wh_ref, bh_ref,
                  out_ref):
    fv = fv_ref[...].astype(jnp.bfloat16)
    x = jnp.dot(fv, w1_ref[...], preferred_element_type=jnp.float32)
    x = jnp.maximum(x + b1_ref[...], 0.0).astype(jnp.bfloat16)
    x = jnp.dot(x, w2_ref[...], preferred_element_type=jnp.float32)
    x = jnp.maximum(x + b2_ref[...], 0.0).astype(jnp.bfloat16)
    out_ref[...] = (
        jnp.dot(x, wh_ref[...], preferred_element_type=jnp.float32)
        + bh_ref[...]
    )


def kernel(feature_vectors, W1, b1, W2, b2, Wc, bc, Wr, br):
    Wh = jnp.concatenate([Wc, Wr], axis=1).astype(jnp.bfloat16)
    bh = jnp.concatenate([bc, br])[None, :]
    out = pl.pallas_call(
        _boxhead_body,
        grid=(pl.cdiv(_N, _TN),),
        in_specs=[
            pl.BlockSpec((_TN, _D), lambda i: (i, 0)),
            pl.BlockSpec((_D, _H), lambda i: (0, 0)),
            pl.BlockSpec((1, _H), lambda i: (0, 0)),
            pl.BlockSpec((_H, _H), lambda i: (0, 0)),
            pl.BlockSpec((1, _H), lambda i: (0, 0)),
            pl.BlockSpec((_H, _O), lambda i: (0, 0)),
            pl.BlockSpec((1, _O), lambda i: (0, 0)),
        ],
        out_specs=pl.BlockSpec((_TN, _O), lambda i: (i, 0)),
        out_shape=jax.ShapeDtypeStruct((_N, _O), jnp.float32),
        compiler_params=pltpu.CompilerParams(
            dimension_semantics=("parallel",),
            vmem_limit_bytes=62 * 1024 * 1024),
    )(feature_vectors, W1.astype(jnp.bfloat16), b1[None, :],
      W2.astype(jnp.bfloat16), b2[None, :], Wh, bh)
    return out[:, :4], out[:, 4:]


# speedup vs baseline: 1.1399x; 1.0280x over previous
import jax
import jax.numpy as jnp
from jax.experimental import pallas as pl
from jax.experimental.pallas import tpu as pltpu

_N = 5000
_D = 12544
_H = 1024
_O = 16
_TN = 1000
_TR = 104
_KT = 7
_DK = _D // _KT


def _boxhead_body(fv_ref, w1_ref, b1_ref, w2_ref, b2_ref, wh_ref, bh_ref,
                  out_ref, w1b_ref, acc_ref):
    n = pl.program_id(0)
    k = pl.program_id(1)
    ksl = pl.ds(k * _DK, _DK)

    @pl.when(n == 0)
    def _cast_w1_block():
        w1b_ref[ksl, :] = w1_ref[...].astype(jnp.bfloat16)

    part = jnp.dot(fv_ref[...], w1b_ref[ksl, :],
                   preferred_element_type=jnp.float32)

    @pl.when(k == 0)
    def _init():
        acc_ref[...] = part

    @pl.when(jnp.logical_and(k > 0, k < _KT - 1))
    def _accum():
        acc_ref[...] = acc_ref[...] + part

    @pl.when(k == _KT - 1)
    def _finish():
        acc_ref[...] = acc_ref[...] + part
        for r in range(-(-_TN // _TR)):
            rs = pl.ds(r * _TR, min(_TR, _TN - r * _TR))
            x = jnp.maximum(acc_ref[rs, :] + b1_ref[...], 0.0)
            x = jnp.dot(x, w2_ref[...], preferred_element_type=jnp.float32)
            x = jnp.maximum(x + b2_ref[...], 0.0)
            out_ref[rs, :] = (
                jnp.dot(x, wh_ref[...], preferred_element_type=jnp.float32)
                + bh_ref[...]
            )


def kernel(feature_vectors, W1, b1, W2, b2, Wc, bc, Wr, br):
    Wh = jnp.concatenate([Wc, Wr], axis=1).astype(jnp.bfloat16)
    bh = jnp.concatenate([bc, br])[None, :]
    out = pl.pallas_call(
        _boxhead_body,
        grid=(pl.cdiv(_N, _TN), _KT),
        in_specs=[
            pl.BlockSpec((_TN, _DK), lambda n, k: (n, k)),
            pl.BlockSpec((_DK, _H),
                         lambda n, k: (jnp.where(n == 0, k, 0), 0)),
            pl.BlockSpec((1, _H), lambda n, k: (0, 0)),
            pl.BlockSpec((_H, _H), lambda n, k: (0, 0)),
            pl.BlockSpec((1, _H), lambda n, k: (0, 0)),
            pl.BlockSpec((_H, _O), lambda n, k: (0, 0)),
            pl.BlockSpec((1, _O), lambda n, k: (0, 0)),
        ],
        out_specs=pl.BlockSpec((_TN, _O), lambda n, k: (n, 0)),
        out_shape=jax.ShapeDtypeStruct((_N, _O), jnp.float32),
        scratch_shapes=[
            pltpu.VMEM((_D, _H), jnp.bfloat16),
            pltpu.VMEM((_TN, _H), jnp.float32),
        ],
        compiler_params=pltpu.CompilerParams(
            vmem_limit_bytes=110 * 1024 * 1024),
    )(feature_vectors, W1, b1[None, :], W2.astype(jnp.bfloat16),
      b2[None, :], Wh, bh)
    return out[:, :4], out[:, 4:]
